# Initial kernel scaffold; baseline (speedup 1.0000x reference)
import jax
import jax.numpy as jnp
from jax.experimental import pallas as pl


def kernel(x, edge_index, W1, b1, W2, b2, W3, b3, W4, b4):
    raise NotImplementedError("write your pallas kernel here")


# speedup vs baseline: 4.0519x; 4.0519x over previous
import functools

import jax
import jax.numpy as jnp
from jax import lax
from jax.experimental import pallas as pl
from jax.experimental.pallas import tpu as pltpu
from jax.experimental.pallas import tpu_sc as plsc

N = 10000
E = 160000
NP = 10240
D_IN = 256
DH = 512
DO = 256
NC = 4
DC = DH // NC
EB = 80
EPT = E // 16
NBLK = EPT // EB
RPT = NP // 16

_f32 = jnp.float32


@functools.cache
def _mesh():
    return plsc.VectorSubcoreMesh(core_axis_name="c", subcore_axis_name="s")


def _deg_body(dst_hbm, zeros16_hbm, out_hbm, idx_v, ones_v, deg_sh):
    cid = lax.axis_index("c")
    tid = lax.axis_index("s")

    @pl.when(cid == 0)
    def _():
        one16 = jnp.ones((16,), _f32)

        def init_ones(i, carry):
            ones_v[i, :] = one16
            return carry

        lax.fori_loop(0, EB, init_ones, 0)
        pltpu.sync_copy(zeros16_hbm.at[pl.ds(tid * RPT, RPT)],
                        deg_sh.at[pl.ds(tid * RPT, RPT)])
        plsc.subcore_barrier()

        def body(j, carry):
            base = tid * EPT + j * EB
            pltpu.sync_copy(dst_hbm.at[pl.ds(base, EB)], idx_v.at[0])
            pltpu.sync_copy(ones_v, deg_sh.at[idx_v.at[0]], add=True)
            return carry

        lax.fori_loop(0, NBLK, body, 0)
        plsc.subcore_barrier()
        pltpu.sync_copy(deg_sh.at[pl.ds(tid * RPT, RPT)],
                        out_hbm.at[pl.ds(tid * RPT, RPT)])


@functools.cache
def _deg_call():
  return pl.kernel(
    _deg_body,
    out_type=jax.ShapeDtypeStruct((NP, 16), _f32),
    mesh=_mesh(),
    scratch_types=[
        pltpu.VMEM((1, EB), jnp.int32),
        pltpu.VMEM((EB, 16), _f32),
        pltpu.VMEM_SHARED((NP, 16), _f32),
    ],
  )


def _spmm_body(h0, h1, h2, h3, src_hbm, dst_hbm, zeros_hbm,
               o0, o1, o2, o3, idx_s, idx_d, gbuf, agg_sh):
    cid = lax.axis_index("c")
    tid = lax.axis_index("s")
    planes = (h0, h1, h2, h3)
    outs = (o0, o1, o2, o3)

    for c in range(NC):
        @pl.when(cid == (c % 2))
        def _(c=c):
            h_p = planes[c]
            o_p = outs[c]
            pltpu.sync_copy(zeros_hbm.at[pl.ds(tid * RPT, RPT)],
                            agg_sh.at[pl.ds(tid * RPT, RPT)])
            plsc.subcore_barrier()

            def body(j, carry):
                base = tid * EPT + j * EB
                pltpu.sync_copy(src_hbm.at[pl.ds(base, EB)], idx_s.at[0])
                pltpu.sync_copy(dst_hbm.at[pl.ds(base, EB)], idx_d.at[0])
                pltpu.sync_copy(h_p.at[idx_s.at[0]], gbuf)
                pltpu.sync_copy(gbuf, agg_sh.at[idx_d.at[0]], add=True)
                return carry

            lax.fori_loop(0, NBLK, body, 0)
            plsc.subcore_barrier()
            pltpu.sync_copy(agg_sh.at[pl.ds(tid * RPT, RPT)],
                            o_p.at[pl.ds(tid * RPT, RPT)])
            plsc.subcore_barrier()


@functools.cache
def _spmm_call():
  return pl.kernel(
    _spmm_body,
    out_type=[jax.ShapeDtypeStruct((NP, DC), _f32) for _ in range(NC)],
    mesh=_mesh(),
    scratch_types=[
        pltpu.VMEM((1, EB), jnp.int32),
        pltpu.VMEM((1, EB), jnp.int32),
        pltpu.VMEM((EB, DC), _f32),
        pltpu.VMEM_SHARED((NP, DC), _f32),
    ],
  )


def _egather_body(a0_hbm, a1_hbm, b0_hbm, b1_hbm, src_hbm, dst_hbm,
                  o_a0, o_a1, o_b0, o_b1, idx_v, rbuf0, rbuf1):
    cid = lax.axis_index("c")
    tid = lax.axis_index("s")
    tabs = ((a0_hbm, a1_hbm), (b0_hbm, b1_hbm))
    idxs = (src_hbm, dst_hbm)
    outs = ((o_a0, o_a1), (o_b0, o_b1))

    for a in range(2):
        @pl.when(cid == a)
        def _(a=a):
            (t0, t1), idxarr, (u0, u1) = tabs[a], idxs[a], outs[a]

            plsc.subcore_barrier()

            def body(j, carry):
                base = tid * EPT + j * EB
                pltpu.sync_copy(idxarr.at[pl.ds(base, EB)], idx_v.at[0])
                pltpu.sync_copy(t0.at[idx_v.at[0]], rbuf0)
                pltpu.sync_copy(t1.at[idx_v.at[0]], rbuf1)
                pltpu.sync_copy(rbuf0, u0.at[pl.ds(base, EB)])
                pltpu.sync_copy(rbuf1, u1.at[pl.ds(base, EB)])
                return carry

            lax.fori_loop(0, NBLK, body, 0)
            plsc.subcore_barrier()


@functools.cache
def _egather_call():
  return pl.kernel(
    _egather_body,
    out_type=[jax.ShapeDtypeStruct((E, DC), _f32) for _ in range(4)],
    mesh=_mesh(),
    scratch_types=[
        pltpu.VMEM((1, EB), jnp.int32),
        pltpu.VMEM((EB, DC), _f32),
        pltpu.VMEM((EB, DC), _f32),
    ],
  )


RB = 320
FB = 640


def _mm1_body(x_ref, w_ref, deg_ref, o0, o1, o2, o3):
    dinv = lax.rsqrt(deg_ref[...] + 1.0)
    h = jnp.dot(x_ref[...], w_ref[...], preferred_element_type=_f32) * dinv
    for j, o in enumerate((o0, o1, o2, o3)):
        o[...] = h[:, j * DC:(j + 1) * DC]


def _mm1(x_p, w1, deg):
    return pl.pallas_call(
        _mm1_body,
        grid=(NP // RB,),
        in_specs=[
            pl.BlockSpec((RB, D_IN), lambda i: (i, 0)),
            pl.BlockSpec((D_IN, DH), lambda i: (0, 0)),
            pl.BlockSpec((RB, 1), lambda i: (i, 0)),
        ],
        out_specs=[pl.BlockSpec((RB, DC), lambda i: (i, 0)) for _ in range(NC)],
        out_shape=[jax.ShapeDtypeStruct((NP, DC), _f32) for _ in range(NC)],
    )(x_p, w1, deg)


def _mm2_body(deg_ref, a0, a1, a2, a3, h0, h1, h2, h3, b_ref, w_ref,
              o0, o1, o2, o3):
    dinv = lax.rsqrt(deg_ref[...] + 1.0)
    agg = jnp.concatenate([a[...] for a in (a0, a1, a2, a3)], axis=1)
    hs = jnp.concatenate([h[...] for h in (h0, h1, h2, h3)], axis=1)
    hmat = jnp.maximum((agg + hs) * dinv + b_ref[...], 0.0)
    out = jnp.dot(hmat, w_ref[...], preferred_element_type=_f32) * dinv
    for j, o in enumerate((o0, o1, o2, o3)):
        o[...] = out[:, j * DC:(j + 1) * DC]


def _mm2(deg, aggs, hss, b1, w2):
    return pl.pallas_call(
        _mm2_body,
        grid=(NP // RB,),
        in_specs=(
            [pl.BlockSpec((RB, 1), lambda i: (i, 0))]
            + [pl.BlockSpec((RB, DC), lambda i: (i, 0)) for _ in range(2 * NC)]
            + [pl.BlockSpec((1, DH), lambda i: (0, 0)),
               pl.BlockSpec((DH, DH), lambda i: (0, 0))]
        ),
        out_specs=[pl.BlockSpec((RB, DC), lambda i: (i, 0)) for _ in range(NC)],
        out_shape=[jax.ShapeDtypeStruct((NP, DC), _f32) for _ in range(NC)],
    )(deg, *aggs, *hss, b1, w2)


def _pack_bf16_pair(even_f32, odd_f32):
    ue = lax.bitcast_convert_type(
        even_f32.astype(jnp.bfloat16).astype(_f32), jnp.uint32)
    uo = lax.bitcast_convert_type(
        odd_f32.astype(jnp.bfloat16).astype(_f32), jnp.uint32)
    w = lax.shift_right_logical(ue, jnp.uint32(16)) | uo
    return lax.bitcast_convert_type(w, jnp.int32)


def _unpack_bf16_pair(w_i32):
    u = lax.bitcast_convert_type(w_i32, jnp.uint32)
    even = lax.bitcast_convert_type(
        lax.shift_left(u, jnp.uint32(16)), _f32)
    odd = lax.bitcast_convert_type(u & jnp.uint32(0xFFFF0000), _f32)
    return even, odd


def _mm3_body(deg_ref, a0, a1, a2, a3, h0, h1, h2, h3, b2_ref, w3ae_ref,
              w3ao_ref, w3be_ref, w3bo_ref, b3e_ref, b3o_ref,
              oa0, oa1, ob0, ob1):
    dinv = lax.rsqrt(deg_ref[...] + 1.0)
    agg = jnp.concatenate([a[...] for a in (a0, a1, a2, a3)], axis=1)
    hs = jnp.concatenate([h[...] for h in (h0, h1, h2, h3)], axis=1)
    h2mat = jnp.maximum((agg + hs) * dinv + b2_ref[...], 0.0)
    ae = jnp.dot(h2mat, w3ae_ref[...], preferred_element_type=_f32)
    ao = jnp.dot(h2mat, w3ao_ref[...], preferred_element_type=_f32)
    be = jnp.dot(h2mat, w3be_ref[...], preferred_element_type=_f32) + b3e_ref[...]
    bo = jnp.dot(h2mat, w3bo_ref[...], preferred_element_type=_f32) + b3o_ref[...]
    wa = _pack_bf16_pair(ae, ao)
    wb = _pack_bf16_pair(be, bo)
    oa0[...] = wa[:, :DC]
    oa1[...] = wa[:, DC:]
    ob0[...] = wb[:, :DC]
    ob1[...] = wb[:, DC:]


def _mm3(deg, aggs, hss, b2, w3ae, w3ao, w3be, w3bo, b3e, b3o):
    return pl.pallas_call(
        _mm3_body,
        grid=(NP // RB,),
        in_specs=(
            [pl.BlockSpec((RB, 1), lambda i: (i, 0))]
            + [pl.BlockSpec((RB, DC), lambda i: (i, 0)) for _ in range(2 * NC)]
            + [pl.BlockSpec((1, DH), lambda i: (0, 0))]
            + [pl.BlockSpec((DH, DH // 2), lambda i: (0, 0)) for _ in range(4)]
            + [pl.BlockSpec((1, DH // 2), lambda i: (0, 0)) for _ in range(2)]
        ),
        out_specs=[pl.BlockSpec((RB, DC), lambda i: (i, 0)) for _ in range(4)],
        out_shape=[jax.ShapeDtypeStruct((NP, DC), jnp.int32) for _ in range(4)],
    )(deg, *aggs, *hss, b2, w3ae, w3ao, w3be, w3bo, b3e, b3o)


def _mm4_body(c1p0, c1p1, c2p0, c2p1, w4e_ref, w4o_ref, b4_ref, o_ref):
    c1 = jnp.concatenate([c1p0[...], c1p1[...]], axis=1)
    c2 = jnp.concatenate([c2p0[...], c2p1[...]], axis=1)
    e1, o1 = _unpack_bf16_pair(c1)
    e2, o2 = _unpack_bf16_pair(c2)
    eh_e = jnp.maximum(e1 + e2, 0.0).astype(jnp.bfloat16)
    eh_o = jnp.maximum(o1 + o2, 0.0).astype(jnp.bfloat16)
    o_ref[...] = (
        jnp.dot(eh_e, w4e_ref[...], preferred_element_type=_f32)
        + jnp.dot(eh_o, w4o_ref[...], preferred_element_type=_f32)
        + b4_ref[...]
    )


def _mm4(c1p0, c1p1, c2p0, c2p1, w4e, w4o, b4):
    return pl.pallas_call(
        _mm4_body,
        grid=(E // FB,),
        in_specs=(
            [pl.BlockSpec((FB, DC), lambda i: (i, 0)) for _ in range(4)]
            + [pl.BlockSpec((DH // 2, DO), lambda i: (0, 0)) for _ in range(2)]
            + [pl.BlockSpec((1, DO), lambda i: (0, 0))]
        ),
        out_specs=pl.BlockSpec((FB, DO), lambda i: (i, 0)),
        out_shape=jax.ShapeDtypeStruct((E, DO), _f32),
    )(c1p0, c1p1, c2p0, c2p1, w4e, w4o, b4)


@jax.jit
def kernel(x, edge_index, W1, b1, W2, b2, W3, b3, W4, b4):
    src = edge_index[0]
    dst = edge_index[1]
    x_p = jnp.pad(x, ((0, NP - N), (0, 0)))
    zeros16 = jnp.zeros((NP, 16), _f32)
    zeros128 = jnp.zeros((NP, DC), _f32)

    deg16 = _deg_call()(dst, zeros16)
    deg = deg16[:, :1]

    hs1 = _mm1(x_p, W1, deg)
    agg1 = _spmm_call()(*hs1, src, dst, zeros128)
    hs2 = _mm2(deg, agg1, hs1, b1.reshape(1, DH), W2)
    agg2 = _spmm_call()(*hs2, src, dst, zeros128)
    w3a, w3b = W3[:DH], W3[DH:]
    planes = _mm3(deg, agg2, hs2, b2.reshape(1, DH),
                  w3a[:, 0::2], w3a[:, 1::2], w3b[:, 0::2], w3b[:, 1::2],
                  b3[0::2].reshape(1, DH // 2), b3[1::2].reshape(1, DH // 2))
    planes_f = [lax.bitcast_convert_type(p, _f32) for p in planes]
    c_planes_f = _egather_call()(*planes_f, src, dst)
    c_planes = [lax.bitcast_convert_type(p, jnp.int32) for p in c_planes_f]
    return _mm4(*c_planes,
                W4[0::2].astype(jnp.bfloat16), W4[1::2].astype(jnp.bfloat16),
                b4.reshape(1, DO))
